# Initial kernel scaffold; baseline (speedup 1.0000x reference)
import jax
import jax.numpy as jnp
from jax.experimental import pallas as pl


def kernel(prob):
    raise NotImplementedError("write your pallas kernel here")


# speedup vs baseline: 2.3900x; 2.3900x over previous
import jax
import jax.numpy as jnp
from jax import lax
from jax.experimental import pallas as pl


_BH = 16


def _disp_block(x):
    c = x.shape[-1]
    m = jnp.max(x, axis=-1)
    iota = lax.broadcasted_iota(jnp.int32, x.shape, 2)
    hit = x == m[..., None]
    idx = jnp.min(jnp.where(hit, iota, c), axis=-1)
    idx_e = idx[..., None]
    low = jnp.sum(jnp.where(iota == idx_e - 1, x, 0.0), axis=-1)
    up = jnp.sum(jnp.where(iota == idx_e + 1, x, 0.0), axis=-1)
    g = jnp.maximum(low, up)
    nbr = jnp.where(up > low, idx + 1, idx - 1).astype(jnp.float32)
    idx_f = idx.astype(jnp.float32)
    conf = m + g
    float_label = (m * idx_f + g * nbr) / conf
    return float_label * jnp.float32(0.035) - jnp.float32(4.0)


def _tc_kernel(prob_ref, out_ref):
    out_ref[...] = _disp_block(prob_ref[...])


def kernel(prob):
    hei, wid, cls = prob.shape
    grid = hei // _BH
    return pl.pallas_call(
        _tc_kernel,
        grid=(grid,),
        in_specs=[pl.BlockSpec((_BH, wid, cls), lambda i: (i, 0, 0))],
        out_specs=pl.BlockSpec((_BH, wid), lambda i: (i, 0)),
        out_shape=jax.ShapeDtypeStruct((hei, wid), jnp.float32),
    )(prob)
